# baseline (speedup 1.0000x reference)
import functools

import jax
import jax.numpy as jnp
from jax import lax
from jax.experimental import pallas as pl
from jax.experimental.pallas import tpu as pltpu
from jax.experimental.pallas import tpu_sc as plsc

_B = 10
_NPER = 1000
_D = 128
_E = 320000
_N = _B * _NPER

_NC = 2
_NS = 16
_NW = _NC * _NS
_CHUNK = 96
_NCHUNKS = 105
_EDGES_PER_TILE = _CHUNK * _NCHUNKS
_EPAD = _NW * _EDGES_PER_TILE
_NPAD = 10240
_ROWS_PER_TILE = _NPAD // _NS
_HROWS = _NPAD // _D

_sc_mesh = plsc.VectorSubcoreMesh(core_axis_name="c", subcore_axis_name="s")


@functools.partial(
    pl.kernel,
    out_type=[
        jax.ShapeDtypeStruct((_NC * _NPAD, _D), jnp.float32),
        jax.ShapeDtypeStruct((_NW * _NPAD,), jnp.float32),
    ],
    mesh=_sc_mesh,
    compiler_params=pltpu.CompilerParams(needs_layout_passes=False),
    scratch_types=[
        pltpu.VMEM_SHARED((_NPAD, _D), jnp.float32),
        pltpu.VMEM((6, _CHUNK), jnp.int32),
        pltpu.VMEM((6, _CHUNK), jnp.int32),
        pltpu.VMEM((_CHUNK, _D), jnp.float32),
        pltpu.VMEM((_CHUNK, _D), jnp.float32),
        pltpu.VMEM((_CHUNK, _D), jnp.float32),
        pltpu.VMEM((_NPAD,), jnp.float32),
        pltpu.SemaphoreType.DMA,
        pltpu.SemaphoreType.DMA,
        pltpu.SemaphoreType.DMA,
        pltpu.SemaphoreType.DMA,
        pltpu.SemaphoreType.DMA,
        pltpu.SemaphoreType.DMA,
        pltpu.SemaphoreType.DMA,
        pltpu.SemaphoreType.DMA,
        pltpu.SemaphoreType.DMA,
        pltpu.SemaphoreType.DMA,
        pltpu.SemaphoreType.DMA,
        pltpu.SemaphoreType.DMA,
    ],
)
def _sc_accumulate(h_hbm, src_hbm, dst_hbm, zf_hbm, z1_hbm,
                   psum_hbm, pcnt_hbm,
                   acc_sp, src_ring, dst_ring, rows0, rows1, rows2, cnt_v,
                   gsem0, gsem1, gsem2, ssem0, ssem1, ssem2,
                   isem0, isem1, isem2, isem3, isem4, isem5):
    cid = lax.axis_index("c")
    sid = lax.axis_index("s")
    rows = (rows0, rows1, rows2)
    gsem = (gsem0, gsem1, gsem2)
    ssem = (ssem0, ssem1, ssem2)
    isem = (isem0, isem1, isem2, isem3, isem4, isem5)

    pltpu.sync_copy(zf_hbm, acc_sp.at[pl.ds(sid * _ROWS_PER_TILE, _ROWS_PER_TILE)])
    pltpu.sync_copy(z1_hbm, cnt_v)
    plsc.subcore_barrier()

    tile_base = (cid * _NS + sid) * _EDGES_PER_TILE
    one16 = jnp.full((16,), 1.0, dtype=jnp.float32)

    def fire_idx(k, b6):
        base = tile_base + k * _CHUNK
        pltpu.async_copy(src_hbm.at[pl.ds(base, _CHUNK)], src_ring.at[b6], isem[b6])
        pltpu.async_copy(dst_hbm.at[pl.ds(base, _CHUNK)], dst_ring.at[b6], isem[b6])

    def wait_idx(k, b6):
        base = tile_base + k * _CHUNK
        pltpu.make_async_copy(
            src_hbm.at[pl.ds(base, _CHUNK)], src_ring.at[b6], isem[b6]).wait()
        pltpu.make_async_copy(
            dst_hbm.at[pl.ds(base, _CHUNK)], dst_ring.at[b6], isem[b6]).wait()

    for j in range(5):
        fire_idx(j, j)
    for b in range(2):
        wait_idx(b, b)
        pltpu.async_copy(h_hbm.at[src_ring.at[b]], rows[b], gsem[b])

    def step(k0, carry):
        for b in range(6):
            k = k0 * 6 + b
            b3 = b % 3
            b6 = b
            bn3 = (b + 2) % 3
            bn6 = (b + 2) % 6
            bp6 = (b + 5) % 6

            @pl.when(k < _NCHUNKS)
            def _():
                pltpu.make_async_copy(
                    h_hbm.at[src_ring.at[b6]], rows[b3], gsem[b3]).wait()
                pltpu.async_copy(
                    rows[b3], acc_sp.at[dst_ring.at[b6]], ssem[b3], add=True)
                for j in range(_CHUNK // 16):
                    d16 = dst_ring[b6, pl.ds(j * 16, 16)]
                    plsc.addupdate_scatter(cnt_v, [d16], one16)

                @pl.when(k >= 1)
                def _():
                    pltpu.make_async_copy(
                        rows[bn3], acc_sp.at[dst_ring.at[bp6]], ssem[bn3]).wait()

                @pl.when(k + 5 < _NCHUNKS)
                def _():
                    fire_idx(k + 5, bp6)

                @pl.when(k + 2 < _NCHUNKS)
                def _():
                    wait_idx(k + 2, bn6)
                    pltpu.async_copy(
                        h_hbm.at[src_ring.at[bn6]], rows[bn3], gsem[bn3])
        return carry

    lax.fori_loop(0, (_NCHUNKS + 5) // 6, step, 0)
    _LB3 = (_NCHUNKS - 1) % 3
    _LB6 = (_NCHUNKS - 1) % 6
    pltpu.make_async_copy(
        rows[_LB3], acc_sp.at[dst_ring.at[_LB6]], ssem[_LB3]).wait()
    plsc.subcore_barrier()

    out_base = cid * _NPAD + sid * _ROWS_PER_TILE
    pltpu.sync_copy(acc_sp.at[pl.ds(sid * _ROWS_PER_TILE, _ROWS_PER_TILE)],
                    psum_hbm.at[pl.ds(out_base, _ROWS_PER_TILE)])
    pltpu.sync_copy(cnt_v, pcnt_hbm.at[pl.ds((cid * _NS + sid) * _NPAD, _NPAD)])


def _combine_body(ps_ref, pc_ref, h_ref, o_ref):
    s = ps_ref[0] + ps_ref[1]
    ones = jnp.ones((_NW, 1), jnp.float32)
    c = lax.dot_general(pc_ref[...], ones, (((0,), (0,)), ((), ())),
                        preferred_element_type=jnp.float32)
    mean = s / jnp.maximum(c, 1.0)
    o_ref[...] = jnp.where(c > 0.0, mean, h_ref[...])


_ROWS_BLK = 1024


def _tc_combine(psum, pcnt_t, h):
    return pl.pallas_call(
        _combine_body,
        grid=(10,),
        in_specs=[
            pl.BlockSpec((_NC, _ROWS_BLK, _D), lambda i: (0, i, 0)),
            pl.BlockSpec((_NW, _ROWS_BLK), lambda i: (0, i)),
            pl.BlockSpec((_ROWS_BLK, _D), lambda i: (i, 0)),
        ],
        out_specs=pl.BlockSpec((_ROWS_BLK, _D), lambda i: (i, 0)),
        out_shape=jax.ShapeDtypeStruct((_N, _D), jnp.float32),
    )(psum, pcnt_t, h)


def kernel(input_features, edge_index):
    h = input_features.reshape(_N, _D)
    pad = _EPAD - _E
    src = jnp.concatenate([edge_index[0], jnp.zeros((pad,), jnp.int32)])
    dst = jnp.concatenate([edge_index[1], jnp.full((pad,), _N, jnp.int32)])
    zf = jnp.zeros((_ROWS_PER_TILE, _D), jnp.float32)
    z1 = jnp.zeros((_NPAD,), jnp.float32)
    psum, pcnt = _sc_accumulate(h, src, dst, zf, z1)
    out = _tc_combine(psum.reshape(_NC, _NPAD, _D), pcnt.reshape(_NW, _NPAD), h)
    return out.reshape(_B, _NPER, _D)


# speedup vs baseline: 1.8698x; 1.8698x over previous
import functools

import jax
import jax.numpy as jnp
from jax import lax
from jax.experimental import pallas as pl
from jax.experimental.pallas import tpu as pltpu
from jax.experimental.pallas import tpu_sc as plsc

_B = 10
_NPER = 1000
_D = 128
_E = 320000
_N = _B * _NPER

_NC = 2
_NS = 16
_NW = _NC * _NS
_EDGES_PER_TILE = _E // _NW
_CHUNK = 80
_NCHUNKS = _EDGES_PER_TILE // _CHUNK
_NPAD = 10240
_ROWS_PER_TILE = _NPAD // _NS
_HROWS = _NPAD // _D

_sc_mesh = plsc.VectorSubcoreMesh(core_axis_name="c", subcore_axis_name="s")


@functools.partial(
    pl.kernel,
    out_type=[
        jax.ShapeDtypeStruct((_NC * _NPAD, _D), jnp.float32),
        jax.ShapeDtypeStruct((_NW * _NPAD,), jnp.float32),
    ],
    mesh=_sc_mesh,
    compiler_params=pltpu.CompilerParams(needs_layout_passes=False),
    scratch_types=[
        pltpu.VMEM_SHARED((_NPAD, _D), jnp.float32),
        pltpu.VMEM((6, _CHUNK), jnp.int32),
        pltpu.VMEM((6, _CHUNK), jnp.int32),
        pltpu.VMEM((_CHUNK, _D), jnp.float32),
        pltpu.VMEM((_CHUNK, _D), jnp.float32),
        pltpu.VMEM((_CHUNK, _D), jnp.float32),
        pltpu.VMEM((_NPAD,), jnp.float32),
        pltpu.SemaphoreType.DMA,
        pltpu.SemaphoreType.DMA,
        pltpu.SemaphoreType.DMA,
        pltpu.SemaphoreType.DMA,
        pltpu.SemaphoreType.DMA,
        pltpu.SemaphoreType.DMA,
        pltpu.SemaphoreType.DMA,
        pltpu.SemaphoreType.DMA,
        pltpu.SemaphoreType.DMA,
        pltpu.SemaphoreType.DMA,
        pltpu.SemaphoreType.DMA,
        pltpu.SemaphoreType.DMA,
    ],
)
def _sc_accumulate(h_hbm, src_hbm, dst_hbm, zf_hbm, z1_hbm,
                   psum_hbm, pcnt_hbm,
                   acc_sp, src_ring, dst_ring, rows0, rows1, rows2, cnt_v,
                   gsem0, gsem1, gsem2, ssem0, ssem1, ssem2,
                   isem0, isem1, isem2, isem3, isem4, isem5):
    cid = lax.axis_index("c")
    sid = lax.axis_index("s")
    rows = (rows0, rows1, rows2)
    gsem = (gsem0, gsem1, gsem2)
    ssem = (ssem0, ssem1, ssem2)
    isem = (isem0, isem1, isem2, isem3, isem4, isem5)

    tile_base = (cid * _NS + sid) * _EDGES_PER_TILE
    one16 = jnp.full((16,), 1.0, dtype=jnp.float32)

    def fire_idx(k, b6):
        base = tile_base + k * _CHUNK
        pltpu.async_copy(src_hbm.at[pl.ds(base, _CHUNK)], src_ring.at[b6], isem[b6])
        pltpu.async_copy(dst_hbm.at[pl.ds(base, _CHUNK)], dst_ring.at[b6], isem[b6])

    def wait_idx(k, b6):
        base = tile_base + k * _CHUNK
        pltpu.make_async_copy(
            src_hbm.at[pl.ds(base, _CHUNK)], src_ring.at[b6], isem[b6]).wait()
        pltpu.make_async_copy(
            dst_hbm.at[pl.ds(base, _CHUNK)], dst_ring.at[b6], isem[b6]).wait()

    zacc = pltpu.async_copy(
        zf_hbm, acc_sp.at[pl.ds(sid * _ROWS_PER_TILE, _ROWS_PER_TILE)], ssem0)
    zcnt = pltpu.async_copy(z1_hbm, cnt_v, ssem1)
    for j in range(5):
        fire_idx(j, j)
    zacc.wait()
    zcnt.wait()
    plsc.subcore_barrier()
    for b in range(2):
        wait_idx(b, b)
        pltpu.async_copy(h_hbm.at[src_ring.at[b]], rows[b], gsem[b])

    def step(k0, carry):
        for b in range(6):
            k = k0 * 6 + b
            b3 = b % 3
            b6 = b
            bn3 = (b + 2) % 3
            bn6 = (b + 2) % 6
            bp6 = (b + 5) % 6

            @pl.when(k < _NCHUNKS)
            def _():
                pltpu.make_async_copy(
                    h_hbm.at[src_ring.at[b6]], rows[b3], gsem[b3]).wait()
                pltpu.async_copy(
                    rows[b3], acc_sp.at[dst_ring.at[b6]], ssem[b3], add=True)

                @pl.when(k >= 1)
                def _():
                    pltpu.make_async_copy(
                        rows[bn3], acc_sp.at[dst_ring.at[bp6]], ssem[bn3]).wait()

                @pl.when(k + 5 < _NCHUNKS)
                def _():
                    fire_idx(k + 5, bp6)

                @pl.when(k + 2 < _NCHUNKS)
                def _():
                    wait_idx(k + 2, bn6)
                    pltpu.async_copy(
                        h_hbm.at[src_ring.at[bn6]], rows[bn3], gsem[bn3])

                for j in range(_CHUNK // 16):
                    d16 = dst_ring[b6, pl.ds(j * 16, 16)]
                    plsc.addupdate_scatter(cnt_v, [d16], one16)
        return carry

    lax.fori_loop(0, (_NCHUNKS + 5) // 6, step, 0)
    _LB3 = (_NCHUNKS - 1) % 3
    _LB6 = (_NCHUNKS - 1) % 6
    pltpu.make_async_copy(
        rows[_LB3], acc_sp.at[dst_ring.at[_LB6]], ssem[_LB3]).wait()
    plsc.subcore_barrier()

    out_base = cid * _NPAD + sid * _ROWS_PER_TILE
    pltpu.sync_copy(acc_sp.at[pl.ds(sid * _ROWS_PER_TILE, _ROWS_PER_TILE)],
                    psum_hbm.at[pl.ds(out_base, _ROWS_PER_TILE)])
    pltpu.sync_copy(cnt_v, pcnt_hbm.at[pl.ds((cid * _NS + sid) * _NPAD, _NPAD)])


def _combine_body(ps_ref, pc_ref, h_ref, o_ref):
    s = ps_ref[0] + ps_ref[1]
    ones = jnp.ones((_NW, 1), jnp.float32)
    c = lax.dot_general(pc_ref[...], ones, (((0,), (0,)), ((), ())),
                        preferred_element_type=jnp.float32)
    mean = s / jnp.maximum(c, 1.0)
    o_ref[...] = jnp.where(c > 0.0, mean, h_ref[...])


_ROWS_BLK = 1024


def _tc_combine(psum, pcnt_t, h):
    return pl.pallas_call(
        _combine_body,
        grid=(10,),
        in_specs=[
            pl.BlockSpec((_NC, _ROWS_BLK, _D), lambda i: (0, i, 0)),
            pl.BlockSpec((_NW, _ROWS_BLK), lambda i: (0, i)),
            pl.BlockSpec((_ROWS_BLK, _D), lambda i: (i, 0)),
        ],
        out_specs=pl.BlockSpec((_ROWS_BLK, _D), lambda i: (i, 0)),
        out_shape=jax.ShapeDtypeStruct((_N, _D), jnp.float32),
    )(psum, pcnt_t, h)


def kernel(input_features, edge_index):
    h = input_features.reshape(_N, _D)
    src = edge_index[0]
    dst = edge_index[1]
    zf = jnp.zeros((_ROWS_PER_TILE, _D), jnp.float32)
    z1 = jnp.zeros((_NPAD,), jnp.float32)
    psum, pcnt = _sc_accumulate(h, src, dst, zf, z1)
    out = _tc_combine(psum.reshape(_NC, _NPAD, _D), pcnt.reshape(_NW, _NPAD), h)
    return out.reshape(_B, _NPER, _D)


# speedup vs baseline: 1.8984x; 1.0153x over previous
import functools

import jax
import jax.numpy as jnp
from jax import lax
from jax.experimental import pallas as pl
from jax.experimental.pallas import tpu as pltpu
from jax.experimental.pallas import tpu_sc as plsc

_B = 10
_NPER = 1000
_D = 128
_E = 320000
_N = _B * _NPER

_NC = 2
_NS = 16
_NW = _NC * _NS
_EDGES_PER_TILE = _E // _NW
_CHUNK = 80
_NCHUNKS = _EDGES_PER_TILE // _CHUNK
_NPAD = 10240
_ROWS_PER_TILE = _NPAD // _NS
_HROWS = _NPAD // _D

_sc_mesh = plsc.VectorSubcoreMesh(core_axis_name="c", subcore_axis_name="s")


@functools.partial(
    pl.kernel,
    out_type=[
        jax.ShapeDtypeStruct((_NC * _NPAD, _D), jnp.float32),
        jax.ShapeDtypeStruct((_NW * _NPAD,), jnp.float32),
    ],
    mesh=_sc_mesh,
    compiler_params=pltpu.CompilerParams(needs_layout_passes=False),
    scratch_types=[
        pltpu.VMEM_SHARED((_NPAD, _D), jnp.float32),
        pltpu.VMEM((6, _CHUNK), jnp.int32),
        pltpu.VMEM((6, _CHUNK), jnp.int32),
        pltpu.VMEM((_CHUNK, _D), jnp.float32),
        pltpu.VMEM((_CHUNK, _D), jnp.float32),
        pltpu.VMEM((_CHUNK, _D), jnp.float32),
        pltpu.VMEM((_NPAD,), jnp.float32),
        pltpu.SemaphoreType.DMA,
        pltpu.SemaphoreType.DMA,
        pltpu.SemaphoreType.DMA,
        pltpu.SemaphoreType.DMA,
        pltpu.SemaphoreType.DMA,
        pltpu.SemaphoreType.DMA,
        pltpu.SemaphoreType.DMA,
        pltpu.SemaphoreType.DMA,
        pltpu.SemaphoreType.DMA,
        pltpu.SemaphoreType.DMA,
        pltpu.SemaphoreType.DMA,
        pltpu.SemaphoreType.DMA,
    ],
)
def _sc_accumulate(h_hbm, src_hbm, dst_hbm, zf_hbm, z1_hbm,
                   psum_hbm, pcnt_hbm,
                   acc_sp, src_ring, dst_ring, rows0, rows1, rows2, cnt_v,
                   gsem0, gsem1, gsem2, ssem0, ssem1, ssem2,
                   isem0, isem1, isem2, isem3, isem4, isem5):
    cid = lax.axis_index("c")
    sid = lax.axis_index("s")
    rows = (rows0, rows1, rows2)
    gsem = (gsem0, gsem1, gsem2)
    ssem = (ssem0, ssem1, ssem2)
    isem = (isem0, isem1, isem2, isem3, isem4, isem5)

    tile_base = (cid * _NS + sid) * _EDGES_PER_TILE
    one16 = jnp.full((16,), 1.0, dtype=jnp.float32)

    def fire_idx(k, b6):
        base = tile_base + k * _CHUNK
        pltpu.async_copy(src_hbm.at[pl.ds(base, _CHUNK)], src_ring.at[b6], isem[b6])
        pltpu.async_copy(dst_hbm.at[pl.ds(base, _CHUNK)], dst_ring.at[b6], isem[b6])

    def wait_idx(k, b6):
        base = tile_base + k * _CHUNK
        pltpu.make_async_copy(
            src_hbm.at[pl.ds(base, _CHUNK)], src_ring.at[b6], isem[b6]).wait()
        pltpu.make_async_copy(
            dst_hbm.at[pl.ds(base, _CHUNK)], dst_ring.at[b6], isem[b6]).wait()

    zacc = pltpu.async_copy(
        zf_hbm, acc_sp.at[pl.ds(sid * _ROWS_PER_TILE, _ROWS_PER_TILE)], ssem0)
    zcnt = pltpu.async_copy(z1_hbm, cnt_v, ssem1)
    for j in range(5):
        fire_idx(j, j)
    zacc.wait()
    zcnt.wait()
    plsc.subcore_barrier()
    for b in range(2):
        wait_idx(b, b)
        pltpu.async_copy(h_hbm.at[src_ring.at[b]], rows[b], gsem[b])

    def step(k0, carry):
        for b in range(6):
            k = k0 * 6 + b
            b3 = b % 3
            b6 = b
            bn3 = (b + 2) % 3
            bn6 = (b + 2) % 6
            bp6 = (b + 5) % 6

            @pl.when(k < _NCHUNKS)
            def _():
                pltpu.make_async_copy(
                    h_hbm.at[src_ring.at[b6]], rows[b3], gsem[b3]).wait()
                pltpu.async_copy(
                    rows[b3], acc_sp.at[dst_ring.at[b6]], ssem[b3], add=True)

                @pl.when(k >= 1)
                def _():
                    pltpu.make_async_copy(
                        rows[bn3], acc_sp.at[dst_ring.at[bp6]], ssem[bn3]).wait()

                @pl.when(k + 5 < _NCHUNKS)
                def _():
                    fire_idx(k + 5, bp6)

                @pl.when(k + 2 < _NCHUNKS)
                def _():
                    wait_idx(k + 2, bn6)
                    pltpu.async_copy(
                        h_hbm.at[src_ring.at[bn6]], rows[bn3], gsem[bn3])

                for j in range(_CHUNK // 16):
                    d16 = dst_ring[b6, pl.ds(j * 16, 16)]
                    plsc.addupdate_scatter(cnt_v, [d16], one16)
        return carry

    lax.fori_loop(0, (_NCHUNKS + 5) // 6, step, 0)
    _LB3 = (_NCHUNKS - 1) % 3
    _LB6 = (_NCHUNKS - 1) % 6
    pltpu.make_async_copy(
        rows[_LB3], acc_sp.at[dst_ring.at[_LB6]], ssem[_LB3]).wait()
    cw = pltpu.async_copy(
        cnt_v, pcnt_hbm.at[pl.ds((cid * _NS + sid) * _NPAD, _NPAD)], gsem0)
    plsc.subcore_barrier()

    out_base = cid * _NPAD + sid * _ROWS_PER_TILE
    pltpu.sync_copy(acc_sp.at[pl.ds(sid * _ROWS_PER_TILE, _ROWS_PER_TILE)],
                    psum_hbm.at[pl.ds(out_base, _ROWS_PER_TILE)])
    cw.wait()


def _combine_body(ps_ref, pc_ref, h_ref, o_ref):
    s = ps_ref[0] + ps_ref[1]
    ones = jnp.ones((_NW, 1), jnp.float32)
    c = lax.dot_general(pc_ref[...], ones, (((0,), (0,)), ((), ())),
                        preferred_element_type=jnp.float32)
    mean = s / jnp.maximum(c, 1.0)
    o_ref[...] = jnp.where(c > 0.0, mean, h_ref[...])


_ROWS_BLK = 2048


def _tc_combine(psum, pcnt_t, h):
    return pl.pallas_call(
        _combine_body,
        grid=(_NPAD // _ROWS_BLK,),
        in_specs=[
            pl.BlockSpec((_NC, _ROWS_BLK, _D), lambda i: (0, i, 0)),
            pl.BlockSpec((_NW, _ROWS_BLK), lambda i: (0, i)),
            pl.BlockSpec((_ROWS_BLK, _D), lambda i: (i, 0)),
        ],
        out_specs=pl.BlockSpec((_ROWS_BLK, _D), lambda i: (i, 0)),
        out_shape=jax.ShapeDtypeStruct((_N, _D), jnp.float32),
    )(psum, pcnt_t, h)


def kernel(input_features, edge_index):
    h = input_features.reshape(_N, _D)
    src = edge_index[0]
    dst = edge_index[1]
    zf = jnp.zeros((_ROWS_PER_TILE, _D), jnp.float32)
    z1 = jnp.zeros((_NPAD,), jnp.float32)
    psum, pcnt = _sc_accumulate(h, src, dst, zf, z1)
    out = _tc_combine(psum.reshape(_NC, _NPAD, _D), pcnt.reshape(_NW, _NPAD), h)
    return out.reshape(_B, _NPER, _D)


# speedup vs baseline: 1.9008x; 1.0013x over previous
import functools

import jax
import jax.numpy as jnp
from jax import lax
from jax.experimental import pallas as pl
from jax.experimental.pallas import tpu as pltpu
from jax.experimental.pallas import tpu_sc as plsc

_B = 10
_NPER = 1000
_D = 128
_E = 320000
_N = _B * _NPER

_NC = 2
_NS = 16
_NW = _NC * _NS
_EDGES_PER_TILE = _E // _NW
_CHUNK = 80
_NCHUNKS = _EDGES_PER_TILE // _CHUNK
_NPAD = 10240
_ROWS_PER_TILE = _NPAD // _NS

_sc_mesh = plsc.VectorSubcoreMesh(core_axis_name="c", subcore_axis_name="s")


@functools.partial(
    pl.kernel,
    out_type=[
        jax.ShapeDtypeStruct((_NC * _NPAD, _D), jnp.float32),
        jax.ShapeDtypeStruct((_NW * _NPAD,), jnp.float32),
    ],
    mesh=_sc_mesh,
    compiler_params=pltpu.CompilerParams(needs_layout_passes=False),
    scratch_types=[
        pltpu.VMEM_SHARED((_NPAD, _D), jnp.float32),
        pltpu.VMEM((6, _CHUNK), jnp.int32),
        pltpu.VMEM((6, _CHUNK), jnp.int32),
        pltpu.VMEM((_CHUNK, _D), jnp.float32),
        pltpu.VMEM((_CHUNK, _D), jnp.float32),
        pltpu.VMEM((_CHUNK, _D), jnp.float32),
        pltpu.VMEM((_NPAD,), jnp.float32),
        pltpu.SemaphoreType.DMA,
        pltpu.SemaphoreType.DMA,
        pltpu.SemaphoreType.DMA,
        pltpu.SemaphoreType.DMA,
        pltpu.SemaphoreType.DMA,
        pltpu.SemaphoreType.DMA,
        pltpu.SemaphoreType.DMA,
        pltpu.SemaphoreType.DMA,
        pltpu.SemaphoreType.DMA,
        pltpu.SemaphoreType.DMA,
        pltpu.SemaphoreType.DMA,
        pltpu.SemaphoreType.DMA,
    ],
)
def _sc_accumulate(h_hbm, src_hbm, dst_hbm, zf_hbm, z1_hbm,
                   psum_hbm, pcnt_hbm,
                   acc_sp, src_ring, dst_ring, rows0, rows1, rows2, cnt_v,
                   gsem0, gsem1, gsem2, ssem0, ssem1, ssem2,
                   isem0, isem1, isem2, isem3, isem4, isem5):
    cid = lax.axis_index("c")
    sid = lax.axis_index("s")
    rows = (rows0, rows1, rows2)
    gsem = (gsem0, gsem1, gsem2)
    ssem = (ssem0, ssem1, ssem2)
    isem = (isem0, isem1, isem2, isem3, isem4, isem5)

    tile_base = (cid * _NS + sid) * _EDGES_PER_TILE
    one16 = jnp.full((16,), 1.0, dtype=jnp.float32)

    def fire_idx(k, b6):
        base = tile_base + k * _CHUNK
        pltpu.async_copy(src_hbm.at[pl.ds(base, _CHUNK)], src_ring.at[b6], isem[b6])
        pltpu.async_copy(dst_hbm.at[pl.ds(base, _CHUNK)], dst_ring.at[b6], isem[b6])

    def wait_idx(k, b6):
        base = tile_base + k * _CHUNK
        pltpu.make_async_copy(
            src_hbm.at[pl.ds(base, _CHUNK)], src_ring.at[b6], isem[b6]).wait()
        pltpu.make_async_copy(
            dst_hbm.at[pl.ds(base, _CHUNK)], dst_ring.at[b6], isem[b6]).wait()

    zacc = pltpu.async_copy(
        zf_hbm, acc_sp.at[pl.ds(sid * _ROWS_PER_TILE, _ROWS_PER_TILE)], ssem0)
    zcnt = pltpu.async_copy(z1_hbm, cnt_v, ssem1)
    for j in range(5):
        fire_idx(j, j)
    zacc.wait()
    zcnt.wait()
    plsc.subcore_barrier()
    for b in range(2):
        wait_idx(b, b)
        pltpu.async_copy(h_hbm.at[src_ring.at[b]], rows[b], gsem[b])

    def step(k0, carry):
        for b in range(6):
            k = k0 * 6 + b
            b3 = b % 3
            b6 = b
            bn3 = (b + 2) % 3
            bn6 = (b + 2) % 6
            bp6 = (b + 5) % 6

            @pl.when(k < _NCHUNKS)
            def _():
                pltpu.make_async_copy(
                    h_hbm.at[src_ring.at[b6]], rows[b3], gsem[b3]).wait()
                pltpu.async_copy(
                    rows[b3], acc_sp.at[dst_ring.at[b6]], ssem[b3], add=True)

                @pl.when(k >= 1)
                def _():
                    pltpu.make_async_copy(
                        rows[bn3], acc_sp.at[dst_ring.at[bp6]], ssem[bn3]).wait()

                @pl.when(k + 5 < _NCHUNKS)
                def _():
                    fire_idx(k + 5, bp6)

                @pl.when(k + 2 < _NCHUNKS)
                def _():
                    wait_idx(k + 2, bn6)
                    pltpu.async_copy(
                        h_hbm.at[src_ring.at[bn6]], rows[bn3], gsem[bn3])

                for j in range(_CHUNK // 16):
                    d16 = dst_ring[b6, pl.ds(j * 16, 16)]
                    plsc.addupdate_scatter(cnt_v, [d16], one16)
        return carry

    lax.fori_loop(0, (_NCHUNKS + 5) // 6, step, 0)
    _LB3 = (_NCHUNKS - 1) % 3
    _LB6 = (_NCHUNKS - 1) % 6
    pltpu.make_async_copy(
        rows[_LB3], acc_sp.at[dst_ring.at[_LB6]], ssem[_LB3]).wait()
    cw = pltpu.async_copy(
        cnt_v, pcnt_hbm.at[pl.ds((cid * _NS + sid) * _NPAD, _NPAD)], gsem0)
    plsc.subcore_barrier()

    out_base = cid * _NPAD + sid * _ROWS_PER_TILE
    pltpu.sync_copy(acc_sp.at[pl.ds(sid * _ROWS_PER_TILE, _ROWS_PER_TILE)],
                    psum_hbm.at[pl.ds(out_base, _ROWS_PER_TILE)])
    cw.wait()


def _combine_body(ps_ref, pc_ref, h_ref, o_ref):
    s = ps_ref[0] + ps_ref[1]
    ones = jnp.ones((_NW, 1), jnp.float32)
    c = lax.dot_general(pc_ref[...], ones, (((0,), (0,)), ((), ())),
                        preferred_element_type=jnp.float32)
    mean = s / jnp.maximum(c, 1.0)
    o_ref[...] = jnp.where(c > 0.0, mean, h_ref[...])


_ROWS_BLK = 2048


def _tc_combine(psum, pcnt_t, h):
    return pl.pallas_call(
        _combine_body,
        grid=(_NPAD // _ROWS_BLK,),
        in_specs=[
            pl.BlockSpec((_NC, _ROWS_BLK, _D), lambda i: (0, i, 0)),
            pl.BlockSpec((_NW, _ROWS_BLK), lambda i: (0, i)),
            pl.BlockSpec((_ROWS_BLK, _D), lambda i: (i, 0)),
        ],
        out_specs=pl.BlockSpec((_ROWS_BLK, _D), lambda i: (i, 0)),
        out_shape=jax.ShapeDtypeStruct((_N, _D), jnp.float32),
    )(psum, pcnt_t, h)


def kernel(input_features, edge_index):
    h = input_features.reshape(_N, _D)
    src = edge_index[0]
    dst = edge_index[1]
    zf = jnp.zeros((_ROWS_PER_TILE, _D), jnp.float32)
    z1 = jnp.zeros((_NPAD,), jnp.float32)
    psum, pcnt = _sc_accumulate(h, src, dst, zf, z1)
    out = _tc_combine(psum.reshape(_NC, _NPAD, _D), pcnt.reshape(_NW, _NPAD), h)
    return out.reshape(_B, _NPER, _D)
